# baseline (speedup 1.0000x reference)
import functools

import jax
import jax.numpy as jnp
from jax import lax
from jax.experimental import pallas as pl
from jax.experimental.pallas import tpu as pltpu
from jax.experimental.pallas import tpu_sc as plsc

VOCAB = 100000
EMBED = 64
MAXLEN = 50
BATCH = 4096
HID = 32

PL_STRIDE = 64
NPAIR = MAXLEN // 2
PWORDS = PL_STRIDE // 2
NC, NS = 2, 16
NW = NC * NS
BPW = BATCH // NW
KSLC = 10
VSLC = VOCAB // KSLC
NSTEP = 5
VBLK = VSLC // NSTEP


def _fold_body(w1_ref, w2_ref, w3_ref, b1_ref, b2_ref, b3_ref,
               veff_ref, cb_ref):
    w23 = jnp.dot(w2_ref[...], w3_ref[...], preferred_element_type=jnp.float32)
    veff_ref[...] = jnp.dot(w1_ref[...], w23,
                            preferred_element_type=jnp.float32)
    c = (jnp.dot(b1_ref[...], w23, preferred_element_type=jnp.float32)
         + jnp.dot(b2_ref[...], w3_ref[...],
                   preferred_element_type=jnp.float32)
         + b3_ref[...])
    cb_ref[...] = c / MAXLEN


def _proj_body(vmat_ref, cb_ref, *refs):
    tbl_refs, p_ref = refs[:KSLC], refs[KSLC]
    for k in range(KSLC):
        res = (jnp.dot(tbl_refs[k][0], vmat_ref[...],
                       preferred_element_type=jnp.float32)
               + cb_ref[...])
        p_ref[k] = res.astype(jnp.bfloat16)


_sc_mesh = plsc.VectorSubcoreMesh(core_axis_name="c", subcore_axis_name="s")


@functools.partial(
    pl.kernel,
    mesh=_sc_mesh,
    out_type=jax.ShapeDtypeStruct((BATCH,), jnp.float32),
    scratch_types=[
        pltpu.VMEM((MAXLEN, BPW), jnp.int32),
        pltpu.VMEM((MAXLEN, BPW), jnp.uint32),
        pltpu.VMEM((BPW,), jnp.float32),
        pltpu.SemaphoreType.DMA,
    ],
)
def _sc_bag(fidx_hbm, p_hbm, out_hbm, idx_v, g_v, res_v, sem):
    wid = lax.axis_index("s") * NC + lax.axis_index("c")
    pltpu.sync_copy(fidx_hbm.at[wid], idx_v)
    cps = [pltpu.async_copy(p_hbm.at[idx_v.at[l]], g_v.at[l], sem)
           for l in range(MAXLEN)]
    for cp in cps:
        cp.wait()
    for ci in range(BPW // 16):
        sl = pl.ds(ci * 16, 16)
        acc = None
        for l in range(MAXLEN):
            w = g_v[l, sl]
            if l % 2 == 0:
                bits = w << jnp.uint32(16)
            else:
                bits = w & jnp.uint32(0xFFFF0000)
            v = lax.bitcast_convert_type(bits, jnp.float32)
            acc = v if acc is None else acc + v
        res_v[sl] = 1.0 / (1.0 + jnp.exp(-acc))
    pltpu.sync_copy(res_v, out_hbm.at[pl.ds(wid * BPW, BPW)])


def kernel(indices, table, W1, b1, W2, b2, W3, b3):
    veff, cb = pl.pallas_call(
        _fold_body,
        out_shape=(jax.ShapeDtypeStruct((MAXLEN * EMBED, 1), jnp.float32),
                   jax.ShapeDtypeStruct((1, 1), jnp.float32)),
    )(W1, W2, W3, b1.reshape(1, HID), b2.reshape(1, HID), b3.reshape(1, 1))

    vmat = jnp.pad(veff.reshape(MAXLEN, EMBED).T,
                   ((0, 0), (0, PL_STRIDE - MAXLEN)))

    tbl8 = table.reshape(KSLC, VSLC, EMBED)
    P = pl.pallas_call(
        _proj_body,
        grid=(NSTEP,),
        in_specs=[
            pl.BlockSpec((EMBED, PL_STRIDE), lambda i: (0, 0)),
            pl.BlockSpec((1, 1), lambda i: (0, 0)),
        ] + [
            pl.BlockSpec((1, VBLK, EMBED), functools.partial(
                lambda i, k: (k, i, 0), k=k))
            for k in range(KSLC)
        ],
        out_specs=pl.BlockSpec((KSLC, VBLK, PL_STRIDE), lambda i: (0, i, 0)),
        out_shape=jax.ShapeDtypeStruct((KSLC, VSLC, PL_STRIDE), jnp.bfloat16),
    )(vmat, cb, *([tbl8] * KSLC))

    p_words = lax.bitcast_convert_type(
        P.reshape(VOCAB * PWORDS, 2), jnp.uint32)
    fidx = (indices * PWORDS
            + (jnp.arange(MAXLEN, dtype=indices.dtype) // 2)[None, :])
    fidx = fidx.reshape(NW, BPW, MAXLEN).transpose(0, 2, 1)

    out = _sc_bag(fidx, p_words)
    return out.reshape(BATCH, 1)


# speedup vs baseline: 20.0680x; 20.0680x over previous
import functools

import jax
import jax.numpy as jnp
from jax import lax
from jax.experimental import pallas as pl
from jax.experimental.pallas import tpu as pltpu
from jax.experimental.pallas import tpu_sc as plsc

VOCAB = 100000
EMBED = 64
MAXLEN = 50
BATCH = 4096
HID = 32

PWORDS = 32
NC, NS = 2, 16
NW = NC * NS
BPW = BATCH // NW
KSLC = 10
VSLC = VOCAB // KSLC
NSTEP = 5
VBLK = VSLC // NSTEP


def _fold_body(w1_ref, w2_ref, w3_ref, b1_ref, b2_ref, b3_ref,
               veff_ref, cb_ref):
    w23 = jnp.dot(w2_ref[...], w3_ref[...], preferred_element_type=jnp.float32)
    veff_ref[...] = jnp.dot(w1_ref[...], w23,
                            preferred_element_type=jnp.float32)
    c = (jnp.dot(b1_ref[...], w23, preferred_element_type=jnp.float32)
         + jnp.dot(b2_ref[...], w3_ref[...],
                   preferred_element_type=jnp.float32)
         + b3_ref[...])
    cb_ref[...] = c / MAXLEN


def _proj_body(vmat_ref, cb_ref, *refs):
    tbl_refs, p_ref = refs[:KSLC], refs[KSLC]
    for k in range(KSLC):
        p32 = (jnp.dot(tbl_refs[k][0], vmat_ref[...],
                       preferred_element_type=jnp.float32)
               + cb_ref[...])
        u = jax.lax.bitcast_convert_type(p32, jnp.uint32) + jnp.uint32(0x8000)
        lo = jax.lax.shift_right_logical(u[:, :PWORDS], jnp.uint32(16))
        hi = u[:, PWORDS:] & jnp.uint32(0xFFFF0000)
        p_ref[k] = jax.lax.bitcast_convert_type(lo | hi, jnp.int32)


_sc_mesh = plsc.VectorSubcoreMesh(core_axis_name="c", subcore_axis_name="s")


@functools.partial(
    pl.kernel,
    mesh=_sc_mesh,
    out_type=jax.ShapeDtypeStruct((BATCH,), jnp.float32),
    scratch_types=[
        pltpu.VMEM((MAXLEN, BPW), jnp.int32),
        pltpu.VMEM((MAXLEN, BPW), jnp.int32),
        pltpu.VMEM((BPW,), jnp.float32),
        pltpu.SemaphoreType.DMA,
    ],
)
def _sc_bag(fidx_hbm, p_hbm, out_hbm, idx_v, g_v, res_v, sem):
    wid = lax.axis_index("s") * NC + lax.axis_index("c")
    pltpu.sync_copy(fidx_hbm.at[wid], idx_v)
    cps = [pltpu.async_copy(p_hbm.at[idx_v.at[l]], g_v.at[l], sem)
           for l in range(MAXLEN)]
    for cp in cps:
        cp.wait()
    for ci in range(BPW // 16):
        sl = pl.ds(ci * 16, 16)
        acc = None
        for l in range(MAXLEN):
            w = g_v[l, sl]
            bits = (jax.lax.shift_left(w, jnp.int32(16)) if l < PWORDS
                    else w & jnp.int32(-65536))
            val = jax.lax.bitcast_convert_type(bits, jnp.float32)
            acc = val if acc is None else acc + val
        res_v[sl] = 1.0 / (1.0 + jnp.exp(-acc))
    pltpu.sync_copy(res_v, out_hbm.at[pl.ds(wid * BPW, BPW)])


def kernel(indices, table, W1, b1, W2, b2, W3, b3):
    veff, cb = pl.pallas_call(
        _fold_body,
        out_shape=(jax.ShapeDtypeStruct((MAXLEN * EMBED, 1), jnp.float32),
                   jax.ShapeDtypeStruct((1, 1), jnp.float32)),
    )(W1, W2, W3, b1.reshape(1, HID), b2.reshape(1, HID), b3.reshape(1, 1))

    vmat = jnp.pad(veff.reshape(MAXLEN, EMBED).T,
                   ((0, 0), (0, 2 * PWORDS - MAXLEN)))

    tbl8 = table.reshape(KSLC, VSLC, EMBED)
    P = pl.pallas_call(
        _proj_body,
        grid=(NSTEP,),
        in_specs=[
            pl.BlockSpec((EMBED, 2 * PWORDS), lambda i: (0, 0)),
            pl.BlockSpec((1, 1), lambda i: (0, 0)),
        ] + [
            pl.BlockSpec((1, VBLK, EMBED), functools.partial(
                lambda i, k: (k, i, 0), k=k))
            for k in range(KSLC)
        ],
        out_specs=pl.BlockSpec((KSLC, VBLK, PWORDS), lambda i: (0, i, 0)),
        out_shape=jax.ShapeDtypeStruct((KSLC, VSLC, PWORDS), jnp.int32),
    )(vmat, cb, *([tbl8] * KSLC))

    off = jnp.arange(MAXLEN, dtype=indices.dtype) % PWORDS
    fidx = indices * PWORDS + off[None, :]
    fidx = fidx.reshape(NW, BPW, MAXLEN).transpose(0, 2, 1)

    out = _sc_bag(fidx, P.reshape(VOCAB * PWORDS))
    return out.reshape(BATCH, 1)


# speedup vs baseline: 20.3677x; 1.0149x over previous
import functools

import jax
import jax.numpy as jnp
from jax import lax
from jax.experimental import pallas as pl
from jax.experimental.pallas import tpu as pltpu
from jax.experimental.pallas import tpu_sc as plsc

VOCAB = 100000
EMBED = 64
MAXLEN = 50
BATCH = 4096
HID = 32

PWORDS = 32
NC, NS = 2, 16
NW = NC * NS
BPW = BATCH // NW
KSLC = 20
VSLC = VOCAB // KSLC
NSTEP = 5
VBLK = VSLC // NSTEP


def _fold_body(w1_ref, w2_ref, w3_ref, b1_ref, b2_ref, b3_ref,
               veff_ref, cb_ref):
    w23 = jnp.dot(w2_ref[...], w3_ref[...], preferred_element_type=jnp.float32)
    veff_ref[...] = jnp.dot(w1_ref[...], w23,
                            preferred_element_type=jnp.float32)
    c = (jnp.dot(b1_ref[...], w23, preferred_element_type=jnp.float32)
         + jnp.dot(b2_ref[...], w3_ref[...],
                   preferred_element_type=jnp.float32)
         + b3_ref[...])
    cb_ref[...] = c / MAXLEN


def _proj_body(vmat_ref, cb_ref, *refs):
    tbl_refs, p_ref = refs[:KSLC], refs[KSLC]
    for k in range(KSLC):
        p32 = (jnp.dot(tbl_refs[k][0], vmat_ref[...],
                       preferred_element_type=jnp.float32)
               + cb_ref[...])
        u = jax.lax.bitcast_convert_type(p32, jnp.uint32) + jnp.uint32(0x8000)
        lo = jax.lax.shift_right_logical(u[:, :PWORDS], jnp.uint32(16))
        hi = u[:, PWORDS:] & jnp.uint32(0xFFFF0000)
        p_ref[k] = jax.lax.bitcast_convert_type(lo | hi, jnp.int32)


_sc_mesh = plsc.VectorSubcoreMesh(core_axis_name="c", subcore_axis_name="s")


@functools.partial(
    pl.kernel,
    mesh=_sc_mesh,
    out_type=jax.ShapeDtypeStruct((BATCH,), jnp.float32),
    scratch_types=[
        pltpu.VMEM((MAXLEN, BPW), jnp.int32),
        pltpu.VMEM((MAXLEN, BPW), jnp.int32),
        pltpu.VMEM((BPW,), jnp.float32),
        pltpu.SemaphoreType.DMA,
    ],
)
def _sc_bag(fidx_hbm, p_hbm, out_hbm, idx_v, g_v, res_v, sem):
    wid = lax.axis_index("s") * NC + lax.axis_index("c")
    pltpu.sync_copy(fidx_hbm.at[wid], idx_v)
    cps = [pltpu.async_copy(p_hbm.at[idx_v.at[l]], g_v.at[l], sem)
           for l in range(MAXLEN)]
    for cp in cps:
        cp.wait()
    for ci in range(BPW // 16):
        sl = pl.ds(ci * 16, 16)
        acc = None
        for l in range(MAXLEN):
            w = g_v[l, sl]
            bits = (jax.lax.shift_left(w, jnp.int32(16)) if l < PWORDS
                    else w & jnp.int32(-65536))
            val = jax.lax.bitcast_convert_type(bits, jnp.float32)
            acc = val if acc is None else acc + val
        res_v[sl] = 1.0 / (1.0 + jnp.exp(-acc))
    pltpu.sync_copy(res_v, out_hbm.at[pl.ds(wid * BPW, BPW)])


def kernel(indices, table, W1, b1, W2, b2, W3, b3):
    veff, cb = pl.pallas_call(
        _fold_body,
        out_shape=(jax.ShapeDtypeStruct((MAXLEN * EMBED, 1), jnp.float32),
                   jax.ShapeDtypeStruct((1, 1), jnp.float32)),
    )(W1, W2, W3, b1.reshape(1, HID), b2.reshape(1, HID), b3.reshape(1, 1))

    vmat = jnp.pad(veff.reshape(MAXLEN, EMBED).T,
                   ((0, 0), (0, 2 * PWORDS - MAXLEN)))

    tbl8 = table.reshape(KSLC, VSLC, EMBED)
    P = pl.pallas_call(
        _proj_body,
        grid=(NSTEP,),
        in_specs=[
            pl.BlockSpec((EMBED, 2 * PWORDS), lambda i: (0, 0)),
            pl.BlockSpec((1, 1), lambda i: (0, 0)),
        ] + [
            pl.BlockSpec((1, VBLK, EMBED), functools.partial(
                lambda i, k: (k, i, 0), k=k))
            for k in range(KSLC)
        ],
        out_specs=pl.BlockSpec((KSLC, VBLK, PWORDS), lambda i: (0, i, 0)),
        out_shape=jax.ShapeDtypeStruct((KSLC, VSLC, PWORDS), jnp.int32),
    )(vmat, cb, *([tbl8] * KSLC))

    off = jnp.arange(MAXLEN, dtype=indices.dtype) % PWORDS
    fidx = indices * PWORDS + off[None, :]
    fidx = fidx.reshape(NW, BPW, MAXLEN).transpose(0, 2, 1)

    out = _sc_bag(fidx, P.reshape(VOCAB * PWORDS))
    return out.reshape(BATCH, 1)


# speedup vs baseline: 20.4193x; 1.0025x over previous
import functools

import jax
import jax.numpy as jnp
from jax import lax
from jax.experimental import pallas as pl
from jax.experimental.pallas import tpu as pltpu
from jax.experimental.pallas import tpu_sc as plsc

VOCAB = 100000
EMBED = 64
MAXLEN = 50
BATCH = 4096
HID = 32

PWORDS = 32
NC, NS = 2, 16
NW = NC * NS
BPW = BATCH // NW
KSLC = 20
VSLC = VOCAB // KSLC
NSTEP = 5
VBLK = VSLC // NSTEP


def _fold_body(w1_ref, w2_ref, w3_ref, b1_ref, b2_ref, b3_ref,
               veff_ref, cb_ref):
    w23 = jnp.dot(w2_ref[...], w3_ref[...], preferred_element_type=jnp.float32)
    veff_ref[...] = jnp.dot(w1_ref[...], w23,
                            preferred_element_type=jnp.float32)
    c = (jnp.dot(b1_ref[...], w23, preferred_element_type=jnp.float32)
         + jnp.dot(b2_ref[...], w3_ref[...],
                   preferred_element_type=jnp.float32)
         + b3_ref[...])
    cb_ref[...] = c / MAXLEN


def _proj_body(vmat_ref, cb_ref, *refs):
    tbl_refs, p_ref = refs[:KSLC], refs[KSLC]
    for k in range(KSLC):
        p32 = (jnp.dot(tbl_refs[k][0], vmat_ref[...],
                       preferred_element_type=jnp.float32)
               + cb_ref[...])
        u = jax.lax.bitcast_convert_type(p32, jnp.uint32) + jnp.uint32(0x8000)
        lo = jax.lax.shift_right_logical(u[:, :PWORDS], jnp.uint32(16))
        hi = u[:, PWORDS:] & jnp.uint32(0xFFFF0000)
        p_ref[k] = jax.lax.bitcast_convert_type(lo | hi, jnp.int32)


_sc_mesh = plsc.VectorSubcoreMesh(core_axis_name="c", subcore_axis_name="s")


@functools.partial(
    pl.kernel,
    mesh=_sc_mesh,
    out_type=jax.ShapeDtypeStruct((BATCH,), jnp.float32),
    scratch_types=[
        pltpu.VMEM((MAXLEN * BPW,), jnp.int32),
        pltpu.VMEM((MAXLEN * BPW,), jnp.int32),
        pltpu.VMEM((BPW,), jnp.float32),
        pltpu.SemaphoreType.DMA,
    ],
)
def _sc_bag(fidx_hbm, p_hbm, out_hbm, idx_v, g_v, res_v, sem):
    wid = lax.axis_index("s") * NC + lax.axis_index("c")
    pltpu.sync_copy(fidx_hbm.at[wid], idx_v)
    pltpu.async_copy(p_hbm.at[idx_v], g_v, sem).wait()
    for ci in range(BPW // 16):
        sl = ci * 16
        acc = None
        for l in range(MAXLEN):
            w = g_v[pl.ds(l * BPW + sl, 16)]
            bits = (jax.lax.shift_left(w, jnp.int32(16)) if l < PWORDS
                    else w & jnp.int32(-65536))
            val = jax.lax.bitcast_convert_type(bits, jnp.float32)
            acc = val if acc is None else acc + val
        res_v[pl.ds(sl, 16)] = 1.0 / (1.0 + jnp.exp(-acc))
    pltpu.sync_copy(res_v, out_hbm.at[pl.ds(wid * BPW, BPW)])


def kernel(indices, table, W1, b1, W2, b2, W3, b3):
    veff, cb = pl.pallas_call(
        _fold_body,
        out_shape=(jax.ShapeDtypeStruct((MAXLEN * EMBED, 1), jnp.float32),
                   jax.ShapeDtypeStruct((1, 1), jnp.float32)),
    )(W1, W2, W3, b1.reshape(1, HID), b2.reshape(1, HID), b3.reshape(1, 1))

    vmat = jnp.pad(veff.reshape(MAXLEN, EMBED).T,
                   ((0, 0), (0, 2 * PWORDS - MAXLEN)))

    tbl8 = table.reshape(KSLC, VSLC, EMBED)
    P = pl.pallas_call(
        _proj_body,
        grid=(NSTEP,),
        in_specs=[
            pl.BlockSpec((EMBED, 2 * PWORDS), lambda i: (0, 0)),
            pl.BlockSpec((1, 1), lambda i: (0, 0)),
        ] + [
            pl.BlockSpec((1, VBLK, EMBED), functools.partial(
                lambda i, k: (k, i, 0), k=k))
            for k in range(KSLC)
        ],
        out_specs=pl.BlockSpec((KSLC, VBLK, PWORDS), lambda i: (0, i, 0)),
        out_shape=jax.ShapeDtypeStruct((KSLC, VSLC, PWORDS), jnp.int32),
    )(vmat, cb, *([tbl8] * KSLC))

    off = jnp.arange(MAXLEN, dtype=indices.dtype) % PWORDS
    fidx = indices * PWORDS + off[None, :]
    fidx = fidx.reshape(NW, BPW, MAXLEN).transpose(0, 2, 1)
    fidx = fidx.reshape(NW, MAXLEN * BPW)

    out = _sc_bag(fidx, P.reshape(VOCAB * PWORDS))
    return out.reshape(BATCH, 1)
